# baseline (speedup 1.0000x reference)
import functools

import jax
import jax.numpy as jnp
from jax import lax
from jax.experimental import pallas as pl
from jax.experimental.pallas import tpu as pltpu
from jax.experimental.pallas import tpu_sc as plsc


def _dense_body(xn, xg, wn, bn, wg, bg, on, og):
    on[...] = jnp.dot(xn[...], wn[...], preferred_element_type=jnp.float32) + bn[...]
    og[...] = jnp.dot(xg[...], wg[...], preferred_element_type=jnp.float32) + bg[...]


def _dense_tokens(X_num, X_geo, W_num, b_num, W_geo, b_geo):
    B, NUM = X_num.shape
    NGEO = X_geo.shape[1]
    D = W_num.shape[1]
    bs = 2048
    out = pl.pallas_call(
        _dense_body,
        grid=(B // bs,),
        in_specs=[
            pl.BlockSpec((bs, NUM), lambda i: (i, 0)),
            pl.BlockSpec((bs, NGEO), lambda i: (i, 0)),
            pl.BlockSpec((NUM, D), lambda i: (0, 0)),
            pl.BlockSpec((1, D), lambda i: (0, 0)),
            pl.BlockSpec((NGEO, D), lambda i: (0, 0)),
            pl.BlockSpec((1, D), lambda i: (0, 0)),
        ],
        out_specs=[
            pl.BlockSpec((bs, D), lambda i: (i, 0)),
            pl.BlockSpec((bs, D), lambda i: (i, 0)),
        ],
        out_shape=[
            jax.ShapeDtypeStruct((B, D), jnp.float32),
            jax.ShapeDtypeStruct((B, D), jnp.float32),
        ],
    )(X_num, X_geo, W_num, b_num.reshape(1, D), W_geo, b_geo.reshape(1, D))
    return out


@functools.lru_cache(maxsize=None)
def _make_sc_tokenizer(B, NCAT, VOCAB, D, NC, NS, L, chunks):
    NT = NCAT + 2
    NW = NC * NS
    RPW = B // NW
    G = RPW // 128
    assert RPW % 128 == 0 and sum(chunks) == NCAT

    mesh = plsc.VectorSubcoreMesh(core_axis_name="c", subcore_axis_name="s")

    @functools.partial(
        pl.kernel,
        out_type=jax.ShapeDtypeStruct((B, NT * D), jnp.float32),
        mesh=mesh,
        compiler_params=pltpu.CompilerParams(use_tc_tiling_on_sc=False),
        scratch_types=[
            pltpu.VMEM((2, RPW), jnp.int32),
            pltpu.VMEM((2, RPW, D), jnp.float32),
            pltpu.VMEM((RPW, D), jnp.float32),
            pltpu.SemaphoreType.DMA,
            pltpu.SemaphoreType.DMA,
            pltpu.SemaphoreType.DMA,
        ],
    )
    def sc_tok(*args):
        tabs = args[:len(chunks)]
        (xcatT, numt, geot, out2,
         idx, rows, dstage, gsem, wsem0, wsem1) = args[len(chunks):]
        assert len(chunks) == 1
        tab = tabs[0]
        wsems = (wsem0, wsem1)
        wid = lax.axis_index("s") * NC + lax.axis_index("c")
        b0 = wid * RPW

        pltpu.sync_copy(numt.at[pl.ds(b0, RPW), :], dstage)
        pltpu.sync_copy(dstage, out2.at[pl.ds(b0, RPW), pl.ds(0, D)])
        pltpu.sync_copy(geot.at[pl.ds(b0, RPW), :], dstage)
        pltpu.sync_copy(dstage, out2.at[pl.ds(b0, RPW), pl.ds((NT - 1) * D, D)])

        pltpu.sync_copy(xcatT.at[0, pl.ds(b0, RPW)], idx.at[0])
        writes = [None, None]
        for f in range(NCAT):
            p = f % 2
            if writes[p] is not None:
                writes[p].wait()
            gathers = [
                pltpu.async_copy(
                    tab.at[f].at[idx.at[p].at[pl.ds(g * 128, 128)]],
                    rows.at[p].at[pl.ds(g * 128, 128), :], gsem)
                for g in range(G)
            ]
            if f + 1 < NCAT:
                pltpu.sync_copy(xcatT.at[f + 1, pl.ds(b0, RPW)],
                                idx.at[1 - p])
            for cp in gathers:
                cp.wait()
            writes[p] = pltpu.async_copy(
                rows.at[p],
                out2.at[pl.ds(b0, RPW), pl.ds((f + 1) * D, D)], wsems[p])
        for wr in writes:
            wr.wait()

    return sc_tok


@functools.lru_cache(maxsize=None)
def _make_table_linearizer(NCAT, VOCAB, D, NC, NS, L):
    NW = NC * NS
    W = 384
    NWIN = VOCAB // W
    TAIL = ((VOCAB - NWIN * W) // 128) * 128
    NF = NCAT * NWIN
    KK = (NF + 2 * NW - 1) // (2 * NW)
    assert W % 128 == 0 and D == 64 and TAIL % 128 == 0 and TAIL > 0

    mesh = plsc.VectorSubcoreMesh(core_axis_name="c", subcore_axis_name="s")

    @functools.partial(
        pl.kernel,
        out_type=jax.ShapeDtypeStruct((NCAT * VOCAB // 2, 2 * D), jnp.float32),
        mesh=mesh,
        compiler_params=pltpu.CompilerParams(use_tc_tiling_on_sc=True,
                                             needs_layout_passes=False),
        scratch_types=[
            pltpu.VMEM((D, W), jnp.float32),
            pltpu.VMEM((D, W), jnp.float32),
            pltpu.VMEM((W // 2, 2 * D), jnp.float32),
            pltpu.VMEM((W // 2, 2 * D), jnp.float32),
            pltpu.VMEM((D, TAIL), jnp.float32),
            pltpu.VMEM((TAIL // 2, 2 * D), jnp.float32),
            pltpu.SemaphoreType.DMA,
            pltpu.SemaphoreType.DMA,
            pltpu.SemaphoreType.DMA,
            pltpu.SemaphoreType.DMA,
        ],
    )
    def lin(tabT, pairs, slabA, slabB, obufA, obufB, tslab, tobuf,
            isemA, isemB, osemA, osemB):
        wid = lax.axis_index("s") * NC + lax.axis_index("c")
        iota16 = lax.iota(jnp.int32, 16)

        def coords(u):
            f = u // NWIN
            wi = lax.rem(u, NWIN)
            return f, wi * W, f * (VOCAB // 2) + wi * (W // 2)

        def start_in(u, slab, isem):
            f, x0, _ = coords(u)
            return pltpu.async_copy(tabT.at[f, :, pl.ds(x0, W)], slab, isem)

        def transpose(slab, obuf, width):
            def xloop(xx, carry):
                row = lax.shift_right_logical(xx, 1)
                colbase = lax.rem(xx, 2) * D
                for d0 in range(0, D, 16):
                    v = plsc.load_gather(
                        slab, [iota16 + d0, jnp.full((16,), xx, jnp.int32)])
                    obuf[row, pl.ds(colbase + d0, 16)] = v
                return carry
            lax.fori_loop(0, width, xloop, 0)

        def do_window(u, slab, obuf, isem, osem, first):
            f, x0, r0 = coords(u)
            pltpu.make_async_copy(tabT.at[f, :, pl.ds(x0, W)], slab,
                                  isem).wait()

            @pl.when(jnp.logical_not(first))
            def _():
                pltpu.make_async_copy(obuf, pairs.at[pl.ds(0, W // 2), :],
                                      osem).wait()

            transpose(slab, obuf, W)
            pltpu.async_copy(obuf, pairs.at[pl.ds(r0, W // 2), :], osem)

        uA0 = wid

        @pl.when(uA0 < NF)
        def _():
            start_in(uA0, slabA, isemA)

        def body(kk, carry):
            uA = wid + 2 * NW * kk
            uB = uA + NW

            @pl.when(uB < NF)
            def _():
                start_in(uB, slabB, isemB)

            @pl.when(uA < NF)
            def _():
                do_window(uA, slabA, obufA, isemA, osemA, kk == 0)

            @pl.when(uA + 2 * NW < NF)
            def _():
                start_in(uA + 2 * NW, slabA, isemA)

            @pl.when(uB < NF)
            def _():
                do_window(uB, slabB, obufB, isemB, osemB, kk == 0)
            return carry

        lax.fori_loop(0, KK, body, 0)
        pltpu.make_async_copy(obufA, pairs.at[pl.ds(0, W // 2), :],
                              osemA).wait()
        pltpu.make_async_copy(obufB, pairs.at[pl.ds(0, W // 2), :],
                              osemB).wait()

        @pl.when(wid < NCAT)
        def _():
            f = wid
            x0 = NWIN * W
            pltpu.sync_copy(tabT.at[f, :, pl.ds(x0, TAIL)], tslab)
            def xloop(xx, carry):
                row = lax.shift_right_logical(xx, 1)
                colbase = lax.rem(xx, 2) * D
                for d0 in range(0, D, 16):
                    v = plsc.load_gather(
                        tslab, [iota16 + d0, jnp.full((16,), xx, jnp.int32)])
                    tobuf[row, pl.ds(colbase + d0, 16)] = v
                return carry
            lax.fori_loop(0, TAIL, xloop, 0)
            pltpu.sync_copy(
                tobuf,
                pairs.at[pl.ds(f * (VOCAB // 2) + NWIN * (W // 2), TAIL // 2), :])

    return lin


def kernel(X_num, X_cat, X_geo, W_num, b_num, tables, W_geo, b_geo):
    B = X_num.shape[0]
    NCAT, VOCAB, D = tables.shape
    try:
        info = plsc.get_sparse_core_info()
        NC, NS, L = info.num_cores, info.num_subcores, info.num_lanes
    except Exception:
        NC, NS, L = 2, 16, 16

    numt, geot = _dense_tokens(X_num, X_geo, W_num, b_num, W_geo, b_geo)
    lin = _make_table_linearizer(NCAT, VOCAB, D, NC, NS, L)
    pairs = lin(jnp.transpose(tables, (0, 2, 1)))
    rem = VOCAB % 128
    if rem:
        patch = tables[:, VOCAB - rem:, :].reshape(NCAT, rem // 2, 2 * D)
        pairs3 = pairs.reshape(NCAT, VOCAB // 2, 2 * D)
        pairs3 = lax.dynamic_update_slice(
            pairs3, patch, (0, (VOCAB - rem) // 2, 0))
        pairs = pairs3
    tab_lin = pairs.reshape(NCAT, VOCAB, D)
    sc_tok = _make_sc_tokenizer(B, NCAT, VOCAB, D, NC, NS, L, (NCAT,))
    out2 = sc_tok(tab_lin, X_cat.T, numt, geot)
    return out2.reshape(B, NCAT + 2, D)
